# baseline (speedup 1.0000x reference)
import functools
import math

import jax
import jax.numpy as jnp
from jax import lax
from jax.experimental import pallas as pl
from jax.experimental.pallas import tpu as pltpu
from jax.experimental.pallas import tpu_sc as plsc

MAXLEN = 2048
D = 512
C = 256
DC = D + C
S = 1024
B = 16
NCORE = 2
NSUB = 16
CH = 128
ZR = 512
RW = 64


def _ts_body(o_ref):
    nt = C // 2
    log_inc = math.log(10000.0) / (nt - 1.0)
    j = lax.broadcasted_iota(jnp.int32, (MAXLEN, nt), 1).astype(jnp.float32)
    p = lax.broadcasted_iota(jnp.int32, (MAXLEN, nt), 0).astype(jnp.float32)
    st = p * jnp.exp(j * (-log_inc))
    o_ref[:, 0:nt] = jnp.sin(st)
    o_ref[:, nt:C] = jnp.cos(st)


def _sc_body(state_hbm, token_hbm, sl_hbm, tl_hbm, ts_hbm,
             out_hbm, len_hbm,
             stage_v, slv, tlv, lnv, ts_sp, zero_sp):
    cid = lax.axis_index("c")
    sid = lax.axis_index("s")
    b = cid * (B // NCORE) + sid // 2
    h = sid % 2

    zero16 = jnp.zeros((16,), jnp.float32)

    def _zero_rows(ref, width, lo, hi):
        def body(r, _):
            for kk in range(width // 16):
                ref[r, pl.ds(kk * 16, 16)] = zero16
            return 0
        lax.fori_loop(lo, hi, body, 0)

    zr_t = ZR // NSUB
    _zero_rows(stage_v, D, 0, zr_t)
    pltpu.sync_copy(stage_v.at[pl.ds(0, zr_t), :],
                    zero_sp.at[pl.ds(sid * zr_t, zr_t), pl.ds(0, D)])
    pltpu.sync_copy(stage_v.at[pl.ds(0, zr_t), pl.ds(0, C)],
                    zero_sp.at[pl.ds(sid * zr_t, zr_t), pl.ds(D, C)])

    @pl.when(sid == 0)
    def _():
        pltpu.sync_copy(ts_hbm, ts_sp)

    pltpu.sync_copy(sl_hbm, slv)
    pltpu.sync_copy(tl_hbm, tlv)
    lanes = lax.broadcasted_iota(jnp.int32, (16,), 0)
    slvec = slv[...]
    tlvec = tlv[...]
    onb = lanes == b
    sl = jnp.max(jnp.where(onb, slvec, 0))
    tl = jnp.max(jnp.where(onb, tlvec, 0))
    ln = jnp.minimum(sl + tl, MAXLEN)

    @pl.when(jnp.logical_and(cid == 0, sid == 0))
    def _():
        lnv[...] = jnp.minimum(slvec + tlvec, MAXLEN)
        pltpu.sync_copy(lnv, len_hbm)

    plsc.subcore_barrier()

    for jj in range(S // CH // 2):
        start = CH * (2 * jj) + CH * h
        @pl.when(start < sl)
        def _(start=start):
            pltpu.sync_copy(state_hbm.at[b, pl.ds(start, CH), :], stage_v)
            pltpu.sync_copy(stage_v,
                            out_hbm.at[b, pl.ds(start, CH), pl.ds(0, D)])

    for jj in range(MAXLEN // CH // 2):
        start = CH * (2 * jj) + CH * h
        @pl.when(start < ln)
        def _(start=start):
            pltpu.sync_copy(ts_sp.at[pl.ds(start, CH), :],
                            out_hbm.at[b, pl.ds(start, CH), pl.ds(D, C)])

    plsc.subcore_barrier()

    for kk in range(S // CH // 2):
        src = CH * (2 * kk) + CH * h
        @pl.when(src < tl)
        def _(src=src):
            pltpu.sync_copy(token_hbm.at[b, pl.ds(src, CH), :], stage_v)
            pltpu.sync_copy(stage_v,
                            out_hbm.at[b, pl.ds(sl + src, CH), pl.ds(0, D)])

    plsc.subcore_barrier()

    rem = MAXLEN - ln
    n512 = rem // ZR
    for j in range(MAXLEN // ZR):
        @pl.when(jnp.logical_and((j % 2) == h, j < n512))
        def _(j=j):
            pltpu.sync_copy(zero_sp,
                            out_hbm.at[b, pl.ds(MAXLEN - ZR * (j + 1), ZR), :])
    r2 = rem - ZR * n512
    e64 = MAXLEN - ZR * n512
    n64 = r2 // RW
    for j in range(ZR // RW - 1):
        @pl.when(jnp.logical_and((j % 2) == h, j < n64))
        def _(j=j):
            pltpu.sync_copy(zero_sp.at[pl.ds(0, RW), :],
                            out_hbm.at[b, pl.ds(e64 - RW * (j + 1), RW), :])
    r3 = r2 - RW * n64
    e8 = e64 - RW * n64
    n8 = r3 // 8
    for j in range(RW // 8 - 1):
        @pl.when(jnp.logical_and((j % 2) == h, j < n8))
        def _(j=j):
            pltpu.sync_copy(zero_sp.at[pl.ds(0, 8), :],
                            out_hbm.at[b, pl.ds(e8 - 8 * (j + 1), 8), :])
    r4 = r3 - 8 * n8
    for j in range(7):
        @pl.when(jnp.logical_and((j % 2) == h, j < r4))
        def _(j=j):
            pltpu.sync_copy(zero_sp.at[pl.ds(0, 1), :],
                            out_hbm.at[b, pl.ds(ln + j, 1), :])


@jax.jit
def kernel(state_seq, state_seq_length, token_seq, token_seq_length):
    ts = pl.pallas_call(
        _ts_body,
        out_shape=jax.ShapeDtypeStruct((MAXLEN, C), jnp.float32),
    )()
    sc = pl.kernel(
        _sc_body,
        out_type=(
            jax.ShapeDtypeStruct((B, MAXLEN, DC), jnp.float32),
            jax.ShapeDtypeStruct((B,), jnp.int32),
        ),
        mesh=plsc.VectorSubcoreMesh(core_axis_name="c", subcore_axis_name="s"),
        compiler_params=pltpu.CompilerParams(
            use_tc_tiling_on_sc=False, needs_layout_passes=False),
        scratch_types=[
            pltpu.VMEM((CH, D), jnp.float32),
            pltpu.VMEM((B,), jnp.int32),
            pltpu.VMEM((B,), jnp.int32),
            pltpu.VMEM((B,), jnp.int32),
            pltpu.VMEM_SHARED((MAXLEN, C), jnp.float32),
            pltpu.VMEM_SHARED((ZR, DC), jnp.float32),
        ],
    )
    out, ln = sc(state_seq, token_seq,
                 state_seq_length.astype(jnp.int32),
                 token_seq_length.astype(jnp.int32), ts)
    return out, ln
